# baseline (speedup 1.0000x reference)
import functools

import jax
import jax.numpy as jnp
from jax import lax
from jax.experimental import pallas as pl
from jax.experimental.pallas import tpu as pltpu
from jax.experimental.pallas import tpu_sc as plsc

_PATCH = 16
_H = 384
_W = 384
_B = 32
_NPTS = 240
_CHUNK = 192
_HALO = _PATCH - 1
_BUF_ROWS = _CHUNK + 2 * _HALO
_BUF = _BUF_ROWS * _W
_NC = 2
_NS = 16


def _sc_mask_body(base_hbm, mask_hbm, base_v, buf_v):
    wid = lax.axis_index("s") * _NC + lax.axis_index("c")
    pltpu.sync_copy(base_hbm.at[wid], base_v)
    lane = lax.iota(jnp.int32, 16)
    ones = jnp.ones((16,), jnp.float32)
    zeros = jnp.zeros((16,), jnp.float32)

    for chunk in range(_H // _CHUNK):
        r0 = chunk * _CHUNK
        lo = (r0 - _HALO) * _W
        hi = (r0 + _CHUNK) * _W

        def init_body(i, _):
            for u in range(8):
                buf_v[pl.ds(i * 128 + u * 16, 16)] = ones
            return 0
        lax.fori_loop(0, _BUF // 128, init_body, 0, unroll=False)

        def pt_body(j, _):
            base = base_v[j]
            valid = jnp.logical_and(base >= lo, base < hi)
            idx = base - lo + lane
            for dx in range(_PATCH):
                plsc.store_scatter(buf_v, [idx + dx * _W], zeros, mask=valid)
            return 0
        lax.fori_loop(0, _NPTS, pt_body, 0, unroll=False)

        pltpu.sync_copy(buf_v.at[pl.ds(_HALO * _W, _CHUNK * _W)],
                        mask_hbm.at[wid, pl.ds(r0 * _W, _CHUNK * _W)])


def _sc_masks(base3):
    mesh = plsc.VectorSubcoreMesh(core_axis_name="c", subcore_axis_name="s")
    return pl.kernel(
        _sc_mask_body,
        out_type=jax.ShapeDtypeStruct((_B, _H * _W), jnp.float32),
        mesh=mesh,
        scratch_types=[
            pltpu.VMEM((_NPTS, 16), jnp.int32),
            pltpu.VMEM((_BUF,), jnp.float32),
        ],
        compiler_params=pltpu.CompilerParams(needs_layout_passes=False),
    )(base3)


def _mul_body(mask_ref, img_ref, out_ref):
    out_ref[...] = img_ref[...] * mask_ref[...][:, None]


def _apply_mask(mask, imgs):
    b, c, h, w = imgs.shape
    return pl.pallas_call(
        _mul_body,
        grid=(b,),
        in_specs=[
            pl.BlockSpec((1, h, w), lambda i: (i, 0, 0)),
            pl.BlockSpec((1, c, h, w), lambda i: (i, 0, 0, 0)),
        ],
        out_specs=pl.BlockSpec((1, c, h, w), lambda i: (i, 0, 0, 0)),
        out_shape=jax.ShapeDtypeStruct(imgs.shape, imgs.dtype),
        compiler_params=pltpu.CompilerParams(
            dimension_semantics=("arbitrary",),
        ),
    )(mask, imgs)


@jax.jit
def kernel(imgs, points_x, points_y):
    b, _, h, w = imgs.shape
    n = points_x.shape[1]
    base = points_x * w + points_y
    base = jnp.pad(base, ((0, 0), (0, _NPTS - n)), constant_values=-(1 << 24))
    base3 = jnp.broadcast_to(base[:, :, None], (b, _NPTS, 16)).astype(jnp.int32)
    mask = _sc_masks(base3).reshape(b, h, w)
    return _apply_mask(mask, imgs)
